# initial kernel scaffold (unmeasured)
import jax
import jax.numpy as jnp
from jax import lax
from jax.experimental import pallas as pl
from jax.experimental.pallas import tpu as pltpu

N_DEV = 32
M = 8192
N = 4096
CHUNK = M // N_DEV


def _mesh_i_order():
    order = []
    for z in range(4):
        for y in range(4):
            xs = (0, 1) if y % 2 == 0 else (1, 0)
            for x in xs:
                order.append((x, y, z))
    return order


def _hamiltonian_cycle():
    cyc = []
    for y in range(4):
        zs = range(4) if y % 2 == 0 else range(3, -1, -1)
        for z in zs:
            cyc.append((0, y, z))
    for y in range(3, -1, -1):
        zs = range(4) if y % 2 == 1 else range(3, -1, -1)
        for z in zs:
            cyc.append((1, y, z))
    return cyc


_MESH = _mesh_i_order()
_CYC = _hamiltonian_cycle()
assert sorted(_MESH) == sorted(_CYC)
for _a, _b in zip(_CYC, _CYC[1:] + _CYC[:1]):
    assert sum(abs(_i - _j) for _i, _j in zip(_a, _b)) == 1, (_a, _b)
_MESH_IDX = {c: i for i, c in enumerate(_MESH)}
RING = [_MESH_IDX[c] for c in _CYC]
INV = [0] * N_DEV
for _p, _d in enumerate(RING):
    INV[_d] = _p


def _gelu(y):
    c0 = 0.7978845608028654
    return 0.5 * y * (1.0 + jnp.tanh(c0 * (y + 0.044715 * y * y * y)))


def kernel(x, w_mat):
    x = x.astype(jnp.bfloat16)
    w_mat = w_mat.astype(jnp.bfloat16)

    my = lax.axis_index("i")
    ring = jnp.asarray(RING, dtype=jnp.int32)
    inv = jnp.asarray(INV, dtype=jnp.int32)
    p = inv[my]
    right = ring[(p + 1) % N_DEV]
    left = ring[(p - 1) % N_DEV]
    sc = jnp.stack([p, left, right]).astype(jnp.int32)

    def body(sc_ref, x_ref, w_ref, out_ref, send_buf, recv_buf, stage_buf,
             send_sems, recv_sems, store_sems, credit_sem):
        p = sc_ref[0]
        left = sc_ref[1]
        right = sc_ref[2]

        barrier = pltpu.get_barrier_semaphore()
        for nbr in (left, right):
            pl.semaphore_signal(
                barrier, inc=1, device_id=(nbr,),
                device_id_type=pl.DeviceIdType.MESH,
            )
        pl.semaphore_wait(barrier, 2)

        def chunk_idx(off):
            return lax.rem(p + (off % N_DEV), N_DEV)

        def partial_chunk(c):
            xs = x_ref[pl.ds(c * CHUNK, CHUNK), :]
            return jnp.dot(xs, w_ref[...], preferred_element_type=jnp.float32)

        stores = [None, None]
        store_n = [0]

        def store_chunk(c, val_f32):
            slot = store_n[0] % 2
            store_n[0] += 1
            if stores[slot] is not None:
                stores[slot].wait()
            stage_buf[slot, :, :] = val_f32
            cp = pltpu.make_async_copy(
                stage_buf.at[slot],
                out_ref.at[pl.ds(c * CHUNK, CHUNK), :],
                store_sems.at[slot],
            )
            cp.start()
            stores[slot] = cp

        send_buf[0, :, :] = partial_chunk(chunk_idx(0)).astype(jnp.bfloat16)

        for g in range(2 * (N_DEV - 1)):
            slot = g % 2
            if g >= 2:
                pl.semaphore_wait(credit_sem, 1)
            rdma = pltpu.make_async_remote_copy(
                src_ref=send_buf.at[slot],
                dst_ref=recv_buf.at[slot],
                send_sem=send_sems.at[slot],
                recv_sem=recv_sems.at[slot],
                device_id=(right,),
                device_id_type=pl.DeviceIdType.MESH,
            )
            rdma.start()
            if g == N_DEV - 1:
                store_chunk(chunk_idx(1),
                            _gelu(send_buf[slot, :, :].astype(jnp.float32)))
            rdma.wait()
            if g < N_DEV - 1:
                c = chunk_idx(-g - 1)
                acc = (recv_buf[slot, :, :].astype(jnp.float32)
                       + partial_chunk(c))
                send_buf[1 - slot, :, :] = acc.astype(jnp.bfloat16)
            else:
                t = g - (N_DEV - 1)
                c = chunk_idx(-t)
                if g < 2 * (N_DEV - 1) - 1:
                    send_buf[1 - slot, :, :] = recv_buf[slot, :, :]
                store_chunk(c, _gelu(recv_buf[slot, :, :].astype(jnp.float32)))
            if g <= 2 * (N_DEV - 1) - 3:
                pl.semaphore_signal(
                    credit_sem, inc=1, device_id=(left,),
                    device_id_type=pl.DeviceIdType.MESH,
                )

        for cp in stores:
            if cp is not None:
                cp.wait()

    return pl.pallas_call(
        body,
        out_shape=jax.ShapeDtypeStruct((M, N), jnp.float32),
        in_specs=[
            pl.BlockSpec(memory_space=pltpu.SMEM),
            pl.BlockSpec(memory_space=pltpu.VMEM),
            pl.BlockSpec(memory_space=pltpu.VMEM),
        ],
        out_specs=pl.BlockSpec(memory_space=pltpu.ANY),
        scratch_shapes=[
            pltpu.VMEM((2, CHUNK, N), jnp.bfloat16),
            pltpu.VMEM((2, CHUNK, N), jnp.bfloat16),
            pltpu.VMEM((2, CHUNK, N), jnp.float32),
            pltpu.SemaphoreType.DMA((2,)),
            pltpu.SemaphoreType.DMA((2,)),
            pltpu.SemaphoreType.DMA((2,)),
            pltpu.SemaphoreType.REGULAR,
        ],
        compiler_params=pltpu.CompilerParams(collective_id=0),
    )(sc, x, w_mat)


# baseline (device time: 1675941 ns/iter reference)
import jax
import jax.numpy as jnp
from jax import lax
from jax.experimental import pallas as pl
from jax.experimental.pallas import tpu as pltpu

N_DEV = 32
M = 8192
N = 4096
CHUNK = M // N_DEV


def _mesh_i_order():
    order = []
    for z in range(4):
        for y in range(4):
            xs = (0, 1) if y % 2 == 0 else (1, 0)
            for x in xs:
                order.append((x, y, z))
    return order


def _hamiltonian_cycle():
    cyc = []
    for y in range(4):
        zs = range(4) if y % 2 == 0 else range(3, -1, -1)
        for z in zs:
            cyc.append((0, y, z))
    for y in range(3, -1, -1):
        zs = range(4) if y % 2 == 1 else range(3, -1, -1)
        for z in zs:
            cyc.append((1, y, z))
    return cyc


_MESH = _mesh_i_order()
_CYC = _hamiltonian_cycle()
assert sorted(_MESH) == sorted(_CYC)
for _a, _b in zip(_CYC, _CYC[1:] + _CYC[:1]):
    assert sum(abs(_i - _j) for _i, _j in zip(_a, _b)) == 1, (_a, _b)
_MESH_IDX = {c: i for i, c in enumerate(_MESH)}
RING = [_MESH_IDX[c] for c in _CYC]
INV = [0] * N_DEV
for _p, _d in enumerate(RING):
    INV[_d] = _p


def _gelu(y):
    c0 = 0.7978845608028654
    return 0.5 * y * (1.0 + jnp.tanh(c0 * (y + 0.044715 * y * y * y)))


def kernel(x, w_mat):
    x = x.astype(jnp.bfloat16)
    w_mat = w_mat.astype(jnp.bfloat16)

    my = lax.axis_index("i")
    ring = jnp.asarray(RING, dtype=jnp.int32)
    inv = jnp.asarray(INV, dtype=jnp.int32)
    p = inv[my]
    right = ring[(p + 1) % N_DEV]
    left = ring[(p - 1) % N_DEV]
    sc = jnp.stack([p, left, right]).astype(jnp.int32)

    def body(sc_ref, x_ref, w_ref, out_ref, send_buf, recv_buf, stage_buf,
             send_sems, recv_sems, store_sems, credit_sem):
        p = sc_ref[0]
        left = sc_ref[1]
        right = sc_ref[2]

        barrier = pltpu.get_barrier_semaphore()
        for nbr in (left, right):
            pl.semaphore_signal(
                barrier, inc=1, device_id=(nbr,),
                device_id_type=pl.DeviceIdType.MESH,
            )
        pl.semaphore_wait(barrier, 2)

        def partial_chunk(c):
            xs = x_ref[pl.ds(c * CHUNK, CHUNK), :]
            return jnp.dot(xs, w_ref[...], preferred_element_type=jnp.float32)

        def ring_rdma(slot):
            return pltpu.make_async_remote_copy(
                src_ref=send_buf.at[slot],
                dst_ref=recv_buf.at[slot],
                send_sem=send_sems.at[slot],
                recv_sem=recv_sems.at[slot],
                device_id=(right,),
                device_id_type=pl.DeviceIdType.MESH,
            )

        def store_copy(sslot, c):
            return pltpu.make_async_copy(
                stage_buf.at[sslot],
                out_ref.at[pl.ds(c * CHUNK, CHUNK), :],
                store_sems.at[sslot],
            )

        def credit_left():
            pl.semaphore_signal(
                credit_sem, inc=1, device_id=(left,),
                device_id_type=pl.DeviceIdType.MESH,
            )

        send_buf[0, :, :] = partial_chunk(p).astype(jnp.bfloat16)

        def rs_step(g, carry):
            slot = lax.rem(g, 2)

            @pl.when(g >= 2)
            def _():
                pl.semaphore_wait(credit_sem, 1)

            rdma = ring_rdma(slot)
            rdma.start()
            rdma.wait()
            c = lax.rem(p + 2 * N_DEV - g - 1, N_DEV)
            acc = recv_buf[slot, :, :].astype(jnp.float32) + partial_chunk(c)
            send_buf[1 - slot, :, :] = acc.astype(jnp.bfloat16)
            credit_left()
            return carry

        lax.fori_loop(0, N_DEV - 1, rs_step, 0)

        def ag_step(t, carry):
            g = t + N_DEV - 1
            slot = lax.rem(g, 2)
            pl.semaphore_wait(credit_sem, 1)
            rdma = ring_rdma(slot)
            rdma.start()

            @pl.when(t == 0)
            def _():
                cown = lax.rem(p + 1, N_DEV)
                stage_buf[0, :, :] = _gelu(
                    send_buf[slot, :, :].astype(jnp.float32))
                store_copy(0, cown).start()

            rdma.wait()
            c = lax.rem(p + 2 * N_DEV - t, N_DEV)

            @pl.when(t < N_DEV - 2)
            def _():
                send_buf[1 - slot, :, :] = recv_buf[slot, :, :]

            sslot = lax.rem(t + 1, 2)

            @pl.when(t >= 1)
            def _():
                cprev = lax.rem(p + 2 * N_DEV - t + 2, N_DEV)
                store_copy(sslot, cprev).wait()

            stage_buf[sslot, :, :] = _gelu(
                recv_buf[slot, :, :].astype(jnp.float32))
            store_copy(sslot, c).start()

            @pl.when(t <= N_DEV - 4)
            def _():
                credit_left()

            return carry

        lax.fori_loop(0, N_DEV - 1, ag_step, 0)

        for tt in (N_DEV - 3, N_DEV - 2):
            k = tt + 1
            c = lax.rem(p + 2 * N_DEV - tt, N_DEV)
            store_copy(k % 2, c).wait()

    return pl.pallas_call(
        body,
        out_shape=jax.ShapeDtypeStruct((M, N), jnp.float32),
        in_specs=[
            pl.BlockSpec(memory_space=pltpu.SMEM),
            pl.BlockSpec(memory_space=pltpu.VMEM),
            pl.BlockSpec(memory_space=pltpu.VMEM),
        ],
        out_specs=pl.BlockSpec(memory_space=pl.ANY),
        scratch_shapes=[
            pltpu.VMEM((2, CHUNK, N), jnp.bfloat16),
            pltpu.VMEM((2, CHUNK, N), jnp.bfloat16),
            pltpu.VMEM((2, CHUNK, N), jnp.float32),
            pltpu.SemaphoreType.DMA((2,)),
            pltpu.SemaphoreType.DMA((2,)),
            pltpu.SemaphoreType.DMA((2,)),
            pltpu.SemaphoreType.REGULAR,
        ],
        compiler_params=pltpu.CompilerParams(collective_id=0),
    )(sc, x, w_mat)
